# baseline (speedup 1.0000x reference)
import functools

import numpy as np

import jax
import jax.numpy as jnp
from jax import lax
from jax.experimental import pallas as pl
from jax.experimental.pallas import tpu as pltpu
from jax.experimental.pallas import tpu_sc as plsc

N_ATOMS = 10000
N_EDGES = 320000
F = 128
N_RBF = 32
N_MODULES = 5
N_INTER = 3
N_OUTRES = 2
N_ATMRES = 2
CUTOFF = 5.0

N_PAD = 10240
NODE_BLK = 1024
EDGE_BLK = 12800
Z_PAD = 256

NC = 2
NS = 16
CHUNK = 40
EDGES_PER_W = N_EDGES // (NC * NS)
N_CHUNKS = EDGES_PER_W // CHUNK
ROWS_PER_S = N_PAD // NS
ZROWS = 16

_LOG2 = 0.6931471805599453


def _ssp(x):
    return jnp.maximum(x, 0.0) + jnp.log(1.0 + jnp.exp(-jnp.abs(x))) - _LOG2


def _dot(a, b):
    return jnp.dot(a, b, preferred_element_type=jnp.float32)


_SL_NP = np.zeros((F, F // 2), np.float32)
_SH_NP = np.zeros((F, F // 2), np.float32)
for _c in range(F // 32):
    for _i in range(16):
        _SL_NP[32 * _c + _i, 16 * _c + _i] = 1.0
        _SH_NP[32 * _c + 16 + _i, 16 * _c + _i] = 1.0


def _embed_body(an_ref, emb_ref, wj_ref, bj_ref, x_ref, xj_ref):
    an = an_ref[...]
    ids = lax.broadcasted_iota(jnp.int32, (1, Z_PAD), 1)
    oh = (an == ids).astype(jnp.float32)
    x = _dot(oh, emb_ref[...])
    x_ref[...] = x
    xj_ref[...] = _dot(_ssp(x), wj_ref[...]) + bj_ref[...]


def _embed_call(an, emb_p, wj0, bj0):
    grid = (N_PAD // NODE_BLK,)
    return pl.pallas_call(
        _embed_body,
        grid=grid,
        in_specs=[
            pl.BlockSpec((NODE_BLK, 1), lambda i: (i, 0)),
            pl.BlockSpec((Z_PAD, F), lambda i: (0, 0)),
            pl.BlockSpec((F, F), lambda i: (0, 0)),
            pl.BlockSpec((1, F), lambda i: (0, 0)),
        ],
        out_specs=[
            pl.BlockSpec((NODE_BLK, F), lambda i: (i, 0)),
            pl.BlockSpec((NODE_BLK, F), lambda i: (i, 0)),
        ],
        out_shape=[
            jax.ShapeDtypeStruct((N_PAD, F), jnp.float32),
            jax.ShapeDtypeStruct((N_PAD, F), jnp.float32),
        ],
    )(an, emb_p, wj0, bj0)


RBF_BLK = 12800


def _rbf_body(rx_ref, ry_ref, rz_ref, gt_ref):
    rx, ry, rz = rx_ref[...], ry_ref[...], rz_ref[...]
    d = jnp.sqrt(rx * rx + ry * ry + rz * rz)
    fc = jnp.where(d < CUTOFF, 0.5 * (jnp.cos(jnp.pi / CUTOFF * d) + 1.0), 0.0)
    width = CUTOFF / (N_RBF - 1)
    coeff = 0.5 / (width * width)
    db = jnp.broadcast_to(d, (N_RBF, RBF_BLK))
    fcb = jnp.broadcast_to(fc, (N_RBF, RBF_BLK))
    centers = lax.broadcasted_iota(jnp.int32, (N_RBF, 1), 0).astype(jnp.float32) * width
    gt_ref[...] = jnp.exp(-coeff * (db - centers) ** 2) * fcb


def _rbf_call(rx, ry, rz):
    grid = (N_EDGES // RBF_BLK,)
    return pl.pallas_call(
        _rbf_body,
        grid=grid,
        in_specs=[pl.BlockSpec((1, RBF_BLK), lambda i: (0, i))] * 3,
        out_specs=pl.BlockSpec((N_RBF, RBF_BLK), lambda i: (0, i)),
        out_shape=jax.ShapeDtypeStruct((N_RBF, N_EDGES), jnp.float32),
    )(rx, ry, rz)


def _gate_body(gt_ref, gl_ref, gh_ref, gate_ref):
    gl = lax.dot_general(gt_ref[...], gl_ref[...], (((0,), (0,)), ((), ())),
                         preferred_element_type=jnp.float32)
    gh = lax.dot_general(gt_ref[...], gh_ref[...], (((0,), (0,)), ((), ())),
                         preferred_element_type=jnp.float32)
    li = lax.bitcast_convert_type(gl, jnp.int32)
    hi = lax.bitcast_convert_type(gh, jnp.int32)
    li = li + 32767 + ((li >> 16) & 1)
    hi = hi + 32767 + ((hi >> 16) & 1)
    gate_ref[...] = ((li >> 16) & 65535) | (hi & jnp.int32(-65536))


def _gate_call(gt, gl, gh):
    grid = (N_EDGES // EDGE_BLK,)
    return pl.pallas_call(
        _gate_body,
        grid=grid,
        in_specs=[
            pl.BlockSpec((N_RBF, EDGE_BLK), lambda i: (0, i)),
            pl.BlockSpec((N_RBF, F // 2), lambda i: (0, 0)),
            pl.BlockSpec((N_RBF, F // 2), lambda i: (0, 0)),
        ],
        out_specs=pl.BlockSpec((EDGE_BLK, F // 2), lambda i: (i, 0)),
        out_shape=jax.ShapeDtypeStruct((N_EDGES, F // 2), jnp.int32),
    )(gt, gl, gh)


def _nodeb_body(with_next, x_ref, msg_ref, sum_ref, wi_ref, bi_ref,
                riw1_ref, rib1_ref, riw2_ref, rib2_ref, u_ref, wm_ref, bm_ref,
                raw1_ref, rab1_ref, raw2_ref, rab2_ref,
                row1_ref, rob1_ref, row2_ref, rob2_ref, wo_ref, bo_ref,
                *rest):
    if with_next:
        wjn_ref, bjn_ref, xo_ref, sumo_ref, xjo_ref = rest
    else:
        xo_ref, sumo_ref = rest
    x = x_ref[...]
    v = _dot(_ssp(x), wi_ref[...]) + bi_ref[...] + msg_ref[0] + msg_ref[1]
    for t in range(N_INTER):
        v = v + _dot(_ssp(_ssp(v) @ riw1_ref[t] + rib1_ref[t]), riw2_ref[t]) \
              + rib2_ref[t]
    v = _ssp(v)
    xn = u_ref[...] * x + _dot(v, wm_ref[...]) + bm_ref[...]
    for t in range(N_ATMRES):
        xn = xn + _dot(_ssp(_ssp(xn) @ raw1_ref[t] + rab1_ref[t]), raw2_ref[t]) \
                + rab2_ref[t]
    o = xn
    for t in range(N_OUTRES):
        o = o + _dot(_ssp(_ssp(o) @ row1_ref[t] + rob1_ref[t]), row2_ref[t]) \
              + rob2_ref[t]
    xo_ref[...] = xn
    sumo_ref[...] = sum_ref[...] + _dot(_ssp(o), wo_ref[...]) + bo_ref[...]
    if with_next:
        xjo_ref[...] = _dot(_ssp(xn), wjn_ref[...]) + bjn_ref[...]


def _nodeb_call(with_next, x, msgp, summ, wi, bi, riw1, rib1, riw2, rib2,
                u_m, wm, bm, raw1, rab1, raw2, rab2, row1, rob1, row2, rob2,
                wo, bo, wjn=None, bjn=None):
    grid = (N_PAD // NODE_BLK,)
    blk = lambda i: (i, 0)
    full2 = lambda i: (0, 0)
    w_spec = pl.BlockSpec((F, F), full2)
    b_spec = pl.BlockSpec((1, F), full2)
    sw_spec = lambda n: pl.BlockSpec((n, F, F), lambda i: (0, 0, 0))
    sb_spec = lambda n: pl.BlockSpec((n, F), full2)
    in_specs = [
        pl.BlockSpec((NODE_BLK, F), blk),
        pl.BlockSpec((NC, NODE_BLK, F), lambda i: (0, i, 0)),
        pl.BlockSpec((NODE_BLK, F), blk),
        w_spec, b_spec,
        sw_spec(N_INTER), sb_spec(N_INTER), sw_spec(N_INTER), sb_spec(N_INTER),
        b_spec, w_spec, b_spec,
        sw_spec(N_ATMRES), sb_spec(N_ATMRES), sw_spec(N_ATMRES), sb_spec(N_ATMRES),
        sw_spec(N_OUTRES), sb_spec(N_OUTRES), sw_spec(N_OUTRES), sb_spec(N_OUTRES),
        w_spec, b_spec,
    ]
    args = [x, msgp, summ, wi, bi, riw1, rib1, riw2, rib2, u_m, wm, bm,
            raw1, rab1, raw2, rab2, row1, rob1, row2, rob2, wo, bo]
    out_specs = [pl.BlockSpec((NODE_BLK, F), blk), pl.BlockSpec((NODE_BLK, F), blk)]
    out_shape = [jax.ShapeDtypeStruct((N_PAD, F), jnp.float32),
                 jax.ShapeDtypeStruct((N_PAD, F), jnp.float32)]
    if with_next:
        in_specs += [w_spec, b_spec]
        args += [wjn, bjn]
        out_specs.append(pl.BlockSpec((NODE_BLK, F), blk))
        out_shape.append(jax.ShapeDtypeStruct((N_PAD, F), jnp.float32))
    return pl.pallas_call(
        functools.partial(_nodeb_body, with_next),
        grid=grid,
        in_specs=in_specs,
        out_specs=out_specs,
        out_shape=out_shape,
    )(*args)


def _sc_msg_body(xj_hbm, gate_hbm, idxc_hbm, out_hbm,
                 idxc_v, r0, r1, r2, g0, g1, g2,
                 msg_sh, jsem, gsem, tsem, ssem):
    cid = lax.axis_index("c")
    sid = lax.axis_index("s")
    rows = (r0, r1, r2)
    gates = (g0, g1, g2)
    w = cid * NS + sid
    base_w = w * EDGES_PER_W

    def _zrow(r, _):
        for c8 in range(F // 16):
            r0[r, pl.ds(c8 * 16, 16)] = jnp.zeros((16,), jnp.float32)
        return 0
    lax.fori_loop(0, CHUNK, _zrow, 0)
    for b in range(ROWS_PER_S // CHUNK):
        pltpu.sync_copy(r0, msg_sh.at[pl.ds(sid * ROWS_PER_S + b * CHUNK, CHUNK)])
    plsc.subcore_barrier()

    base_c = w * N_CHUNKS

    def issue_idx(c, slot):
        pltpu.async_copy(idxc_hbm.at[base_c + c], idxc_v.at[slot],
                         jsem.at[slot])

    def wait_idx(c, slot):
        pltpu.make_async_copy(idxc_hbm.at[base_c + c], idxc_v.at[slot],
                              jsem.at[slot]).wait()

    def issue_loads(c, slot):
        pltpu.async_copy(xj_hbm.at[idxc_v.at[slot, 0]], rows[slot],
                         gsem.at[slot])
        pltpu.async_copy(gate_hbm.at[pl.ds(base_w + c * CHUNK, CHUNK)],
                         gates[slot], tsem.at[slot])

    def wait_loads(c, slot):
        pltpu.make_async_copy(xj_hbm.at[idxc_v.at[slot, 0]], rows[slot],
                              gsem.at[slot]).wait()
        pltpu.make_async_copy(gate_hbm.at[pl.ds(base_w + c * CHUNK, CHUNK)],
                              gates[slot], tsem.at[slot]).wait()

    def mul_scatter(c, slot):
        def _mrow(r, _):
            for c16 in range(F // 32):
                gw = gates[slot][r, pl.ds(c16 * 16, 16)]
                ge = lax.bitcast_convert_type(gw << 16, jnp.float32)
                go = lax.bitcast_convert_type(gw & jnp.int32(-65536),
                                              jnp.float32)
                sa = pl.ds(c16 * 32, 16)
                sb = pl.ds(c16 * 32 + 16, 16)
                rows[slot][r, sa] = rows[slot][r, sa] * ge
                rows[slot][r, sb] = rows[slot][r, sb] * go
            return 0
        lax.fori_loop(0, CHUNK, _mrow, 0)
        pltpu.async_copy(rows[slot], msg_sh.at[idxc_v.at[slot, 1]], ssem.at[slot],
                         add=True)

    def wait_scatter(c, slot):
        pltpu.make_async_copy(rows[slot], msg_sh.at[idxc_v.at[slot, 1]],
                              ssem.at[slot]).wait()

    issue_idx(0, 0)
    issue_idx(1, 1)
    wait_idx(0, 0)
    issue_loads(0, 0)

    def _outer(i, _):
        for b in range(3):
            c = 3 * i + b
            if b == 0:
                @pl.when(c > 0)
                def _():
                    wait_scatter(c - 1, 2)
            else:
                wait_scatter(c - 1, (b + 2) % 3)

            @pl.when(c + 2 < N_CHUNKS)
            def _():
                issue_idx(c + 2, (b + 2) % 3)
            wait_idx(c + 1, (b + 1) % 3)
            issue_loads(c + 1, (b + 1) % 3)
            wait_loads(c, b)
            mul_scatter(c, b)
        return 0
    lax.fori_loop(0, (N_CHUNKS - 1) // 3, _outer, 0)

    tc = N_CHUNKS - 1
    wait_scatter(tc - 1, 2)
    wait_loads(tc, 0)
    mul_scatter(tc, 0)
    wait_scatter(tc, 0)

    plsc.subcore_barrier()
    pltpu.sync_copy(msg_sh.at[pl.ds(sid * ROWS_PER_S, ROWS_PER_S)],
                    out_hbm.at[cid, pl.ds(sid * ROWS_PER_S, ROWS_PER_S)])


_SC_CALL_CACHE = []


def _sc_msg_call(xj, gate, idx_j, idx_i):
    if not _SC_CALL_CACHE:
        _SC_CALL_CACHE.append(functools.partial(
            pl.kernel,
            out_type=jax.ShapeDtypeStruct((NC, N_PAD, F), jnp.float32),
            mesh=plsc.VectorSubcoreMesh(core_axis_name="c",
                                        subcore_axis_name="s",
                                        num_cores=NC, num_subcores=NS),
            scratch_types=[
                pltpu.VMEM((3, 2, CHUNK), jnp.int32),
                pltpu.VMEM((CHUNK, F), jnp.float32),
                pltpu.VMEM((CHUNK, F), jnp.float32),
                pltpu.VMEM((CHUNK, F), jnp.float32),
                pltpu.VMEM((CHUNK, F // 2), jnp.int32),
                pltpu.VMEM((CHUNK, F // 2), jnp.int32),
                pltpu.VMEM((CHUNK, F // 2), jnp.int32),
                pltpu.VMEM_SHARED((N_PAD, F), jnp.float32),
                pltpu.SemaphoreType.DMA((3,)),
                pltpu.SemaphoreType.DMA((3,)),
                pltpu.SemaphoreType.DMA((3,)),
                pltpu.SemaphoreType.DMA((3,)),
            ],
        )(_sc_msg_body))
    idxc = jnp.stack([idx_j.reshape(N_EDGES // CHUNK, CHUNK),
                      idx_i.reshape(N_EDGES // CHUNK, CHUNK)], axis=1)
    return _SC_CALL_CACHE[0](xj, gate, idxc)


def kernel(atomic_numbers, Rij, idx_i, idx_j, emb, Wi, bi, Wj, bj, G, u,
           Wm, bm, ri_W1, ri_b1, ri_W2, ri_b2, ra_W1, ra_b1, ra_W2, ra_b2,
           ro_W1, ro_b1, ro_W2, ro_b2, Wo, bo):
    an = jnp.pad(atomic_numbers.astype(jnp.int32),
                 (0, N_PAD - N_ATOMS)).reshape(N_PAD, 1)
    emb_p = jnp.pad(emb.astype(jnp.float32), ((0, Z_PAD - emb.shape[0]), (0, 0)))
    idx_i = idx_i.astype(jnp.int32)
    idx_j = idx_j.astype(jnp.int32)
    rij = Rij.astype(jnp.float32)
    rx = rij[:, 0].reshape(1, N_EDGES)
    ry = rij[:, 1].reshape(1, N_EDGES)
    rz = rij[:, 2].reshape(1, N_EDGES)
    gt = _rbf_call(rx, ry, rz)

    g_lo = jnp.einsum("mij,jk->mik", G, jnp.asarray(_SL_NP))
    g_hi = jnp.einsum("mij,jk->mik", G, jnp.asarray(_SH_NP))
    x, xj = _embed_call(an, emb_p, Wj[0], bj[0].reshape(1, F))
    summ = jnp.zeros((N_PAD, F), jnp.float32)
    for m in range(N_MODULES):
        gate = _gate_call(gt, g_lo[m], g_hi[m])
        msgp = _sc_msg_call(xj, gate, idx_j, idx_i)
        args = (x, msgp, summ, Wi[m], bi[m].reshape(1, F),
                ri_W1[m], ri_b1[m], ri_W2[m], ri_b2[m],
                u[m].reshape(1, F), Wm[m], bm[m].reshape(1, F),
                ra_W1[m], ra_b1[m], ra_W2[m], ra_b2[m],
                ro_W1[m], ro_b1[m], ro_W2[m], ro_b2[m],
                Wo[m], bo[m].reshape(1, F))
        if m + 1 < N_MODULES:
            x, summ, xj = _nodeb_call(True, *args,
                                      wjn=Wj[m + 1], bjn=bj[m + 1].reshape(1, F))
        else:
            x, summ = _nodeb_call(False, *args)
    return summ[:N_ATOMS]


# speedup vs baseline: 1.0136x; 1.0136x over previous
import functools

import numpy as np

import jax
import jax.numpy as jnp
from jax import lax
from jax.experimental import pallas as pl
from jax.experimental.pallas import tpu as pltpu
from jax.experimental.pallas import tpu_sc as plsc

N_ATOMS = 10000
N_EDGES = 320000
F = 128
N_RBF = 32
N_MODULES = 5
N_INTER = 3
N_OUTRES = 2
N_ATMRES = 2
CUTOFF = 5.0

N_PAD = 10240
NODE_BLK = 1024
EDGE_BLK = 12800
Z_PAD = 256

NC = 2
NS = 16
CHUNK = 40
EDGES_PER_W = N_EDGES // (NC * NS)
N_CHUNKS = EDGES_PER_W // CHUNK
ROWS_PER_S = N_PAD // NS
ZROWS = 16

_LOG2 = 0.6931471805599453


def _ssp(x):
    return jnp.maximum(x, 0.0) + jnp.log(1.0 + jnp.exp(-jnp.abs(x))) - _LOG2


def _dot(a, b):
    return jnp.dot(a, b, preferred_element_type=jnp.float32)


_SL_NP = np.zeros((F, F // 2), np.float32)
_SH_NP = np.zeros((F, F // 2), np.float32)
for _c in range(F // 32):
    for _i in range(16):
        _SL_NP[32 * _c + _i, 16 * _c + _i] = 1.0
        _SH_NP[32 * _c + 16 + _i, 16 * _c + _i] = 1.0


def _embed_body(an_ref, emb_ref, wj_ref, bj_ref, x_ref, xj_ref):
    an = an_ref[...]
    ids = lax.broadcasted_iota(jnp.int32, (1, Z_PAD), 1)
    oh = (an == ids).astype(jnp.float32)
    x = _dot(oh, emb_ref[...])
    x_ref[...] = x
    xj_ref[...] = _dot(_ssp(x), wj_ref[...]) + bj_ref[...]


def _embed_call(an, emb_p, wj0, bj0):
    grid = (N_PAD // NODE_BLK,)
    return pl.pallas_call(
        _embed_body,
        grid=grid,
        in_specs=[
            pl.BlockSpec((NODE_BLK, 1), lambda i: (i, 0)),
            pl.BlockSpec((Z_PAD, F), lambda i: (0, 0)),
            pl.BlockSpec((F, F), lambda i: (0, 0)),
            pl.BlockSpec((1, F), lambda i: (0, 0)),
        ],
        out_specs=[
            pl.BlockSpec((NODE_BLK, F), lambda i: (i, 0)),
            pl.BlockSpec((NODE_BLK, F), lambda i: (i, 0)),
        ],
        out_shape=[
            jax.ShapeDtypeStruct((N_PAD, F), jnp.float32),
            jax.ShapeDtypeStruct((N_PAD, F), jnp.float32),
        ],
    )(an, emb_p, wj0, bj0)


RBF_BLK = 12800


def _rbf_body(rx_ref, ry_ref, rz_ref, gt_ref):
    rx, ry, rz = rx_ref[...], ry_ref[...], rz_ref[...]
    d = jnp.sqrt(rx * rx + ry * ry + rz * rz)
    fc = jnp.where(d < CUTOFF, 0.5 * (jnp.cos(jnp.pi / CUTOFF * d) + 1.0), 0.0)
    width = CUTOFF / (N_RBF - 1)
    coeff = 0.5 / (width * width)
    db = jnp.broadcast_to(d, (N_RBF, RBF_BLK))
    fcb = jnp.broadcast_to(fc, (N_RBF, RBF_BLK))
    centers = lax.broadcasted_iota(jnp.int32, (N_RBF, 1), 0).astype(jnp.float32) * width
    gt_ref[...] = jnp.exp(-coeff * (db - centers) ** 2) * fcb


def _rbf_call(rx, ry, rz):
    grid = (N_EDGES // RBF_BLK,)
    return pl.pallas_call(
        _rbf_body,
        grid=grid,
        in_specs=[pl.BlockSpec((1, RBF_BLK), lambda i: (0, i))] * 3,
        out_specs=pl.BlockSpec((N_RBF, RBF_BLK), lambda i: (0, i)),
        out_shape=jax.ShapeDtypeStruct((N_RBF, N_EDGES), jnp.float32),
    )(rx, ry, rz)


def _gate_body(gt_ref, gl_ref, gh_ref, gate_ref):
    gl = lax.dot_general(gt_ref[...], gl_ref[...], (((0,), (0,)), ((), ())),
                         preferred_element_type=jnp.float32)
    gh = lax.dot_general(gt_ref[...], gh_ref[...], (((0,), (0,)), ((), ())),
                         preferred_element_type=jnp.float32)
    li = lax.bitcast_convert_type(gl, jnp.int32)
    hi = lax.bitcast_convert_type(gh, jnp.int32)
    li = li + 32767 + ((li >> 16) & 1)
    hi = hi + 32767 + ((hi >> 16) & 1)
    gate_ref[...] = ((li >> 16) & 65535) | (hi & jnp.int32(-65536))


def _gate_call(gt, gl, gh):
    grid = (N_EDGES // EDGE_BLK,)
    return pl.pallas_call(
        _gate_body,
        grid=grid,
        in_specs=[
            pl.BlockSpec((N_RBF, EDGE_BLK), lambda i: (0, i)),
            pl.BlockSpec((N_RBF, F // 2), lambda i: (0, 0)),
            pl.BlockSpec((N_RBF, F // 2), lambda i: (0, 0)),
        ],
        out_specs=pl.BlockSpec((EDGE_BLK, F // 2), lambda i: (i, 0)),
        out_shape=jax.ShapeDtypeStruct((N_EDGES, F // 2), jnp.int32),
    )(gt, gl, gh)


def _nodeb_body(with_next, x_ref, msg_ref, sum_ref, wi_ref, bi_ref,
                riw1_ref, rib1_ref, riw2_ref, rib2_ref, u_ref, wm_ref, bm_ref,
                raw1_ref, rab1_ref, raw2_ref, rab2_ref,
                row1_ref, rob1_ref, row2_ref, rob2_ref, wo_ref, bo_ref,
                *rest):
    if with_next:
        wjn_ref, bjn_ref, xo_ref, sumo_ref, xjo_ref = rest
    else:
        xo_ref, sumo_ref = rest
    x = x_ref[...]
    v = _dot(_ssp(x), wi_ref[...]) + bi_ref[...] + msg_ref[0] + msg_ref[1]
    for t in range(N_INTER):
        v = v + _dot(_ssp(_ssp(v) @ riw1_ref[t] + rib1_ref[t]), riw2_ref[t]) \
              + rib2_ref[t]
    v = _ssp(v)
    xn = u_ref[...] * x + _dot(v, wm_ref[...]) + bm_ref[...]
    for t in range(N_ATMRES):
        xn = xn + _dot(_ssp(_ssp(xn) @ raw1_ref[t] + rab1_ref[t]), raw2_ref[t]) \
                + rab2_ref[t]
    o = xn
    for t in range(N_OUTRES):
        o = o + _dot(_ssp(_ssp(o) @ row1_ref[t] + rob1_ref[t]), row2_ref[t]) \
              + rob2_ref[t]
    xo_ref[...] = xn
    sumo_ref[...] = sum_ref[...] + _dot(_ssp(o), wo_ref[...]) + bo_ref[...]
    if with_next:
        xjo_ref[...] = _dot(_ssp(xn), wjn_ref[...]) + bjn_ref[...]


def _nodeb_call(with_next, x, msgp, summ, wi, bi, riw1, rib1, riw2, rib2,
                u_m, wm, bm, raw1, rab1, raw2, rab2, row1, rob1, row2, rob2,
                wo, bo, wjn=None, bjn=None):
    grid = (N_PAD // NODE_BLK,)
    blk = lambda i: (i, 0)
    full2 = lambda i: (0, 0)
    w_spec = pl.BlockSpec((F, F), full2)
    b_spec = pl.BlockSpec((1, F), full2)
    sw_spec = lambda n: pl.BlockSpec((n, F, F), lambda i: (0, 0, 0))
    sb_spec = lambda n: pl.BlockSpec((n, F), full2)
    in_specs = [
        pl.BlockSpec((NODE_BLK, F), blk),
        pl.BlockSpec((NC, NODE_BLK, F), lambda i: (0, i, 0)),
        pl.BlockSpec((NODE_BLK, F), blk),
        w_spec, b_spec,
        sw_spec(N_INTER), sb_spec(N_INTER), sw_spec(N_INTER), sb_spec(N_INTER),
        b_spec, w_spec, b_spec,
        sw_spec(N_ATMRES), sb_spec(N_ATMRES), sw_spec(N_ATMRES), sb_spec(N_ATMRES),
        sw_spec(N_OUTRES), sb_spec(N_OUTRES), sw_spec(N_OUTRES), sb_spec(N_OUTRES),
        w_spec, b_spec,
    ]
    args = [x, msgp, summ, wi, bi, riw1, rib1, riw2, rib2, u_m, wm, bm,
            raw1, rab1, raw2, rab2, row1, rob1, row2, rob2, wo, bo]
    out_specs = [pl.BlockSpec((NODE_BLK, F), blk), pl.BlockSpec((NODE_BLK, F), blk)]
    out_shape = [jax.ShapeDtypeStruct((N_PAD, F), jnp.float32),
                 jax.ShapeDtypeStruct((N_PAD, F), jnp.float32)]
    if with_next:
        in_specs += [w_spec, b_spec]
        args += [wjn, bjn]
        out_specs.append(pl.BlockSpec((NODE_BLK, F), blk))
        out_shape.append(jax.ShapeDtypeStruct((N_PAD, F), jnp.float32))
    return pl.pallas_call(
        functools.partial(_nodeb_body, with_next),
        grid=grid,
        in_specs=in_specs,
        out_specs=out_specs,
        out_shape=out_shape,
    )(*args)


def _sc_msg_body(xj_hbm, gate_hbm, idxj_hbm, idxi_hbm, out_hbm,
                 idxj_v, idxi_v, r0, r1, r2, g0, g1, g2, p0, p1, p2,
                 zbuf_v, msg_sh, jsem, isem, gsem, tsem, ssem):
    cid = lax.axis_index("c")
    sid = lax.axis_index("s")
    rows = (r0, r1, r2)
    gates = (g0, g1, g2)
    prods = (p0, p1, p2)
    w = cid * NS + sid
    base_w = w * EDGES_PER_W

    def _zrow(r, _):
        for c8 in range(F // 16):
            zbuf_v[r, pl.ds(c8 * 16, 16)] = jnp.zeros((16,), jnp.float32)
        return 0
    lax.fori_loop(0, ZROWS, _zrow, 0)
    for b in range(ROWS_PER_S // ZROWS):
        pltpu.sync_copy(zbuf_v, msg_sh.at[pl.ds(sid * ROWS_PER_S + b * ZROWS, ZROWS)])
    plsc.subcore_barrier()

    def issue_idx(c, slot):
        pltpu.async_copy(idxj_hbm.at[pl.ds(base_w + c * CHUNK, CHUNK)],
                         idxj_v.at[slot], jsem.at[slot])
        pltpu.async_copy(idxi_hbm.at[pl.ds(base_w + c * CHUNK, CHUNK)],
                         idxi_v.at[slot], isem.at[slot])

    def wait_idx(c, slot):
        pltpu.make_async_copy(idxj_hbm.at[pl.ds(base_w + c * CHUNK, CHUNK)],
                              idxj_v.at[slot], jsem.at[slot]).wait()
        pltpu.make_async_copy(idxi_hbm.at[pl.ds(base_w + c * CHUNK, CHUNK)],
                              idxi_v.at[slot], isem.at[slot]).wait()

    def issue_loads(c, slot):
        pltpu.async_copy(xj_hbm.at[idxj_v.at[slot]], rows[slot], gsem.at[slot])
        pltpu.async_copy(gate_hbm.at[pl.ds(base_w + c * CHUNK, CHUNK)],
                         gates[slot], tsem.at[slot])

    def wait_loads(c, slot):
        pltpu.make_async_copy(xj_hbm.at[idxj_v.at[slot]], rows[slot],
                              gsem.at[slot]).wait()
        pltpu.make_async_copy(gate_hbm.at[pl.ds(base_w + c * CHUNK, CHUNK)],
                              gates[slot], tsem.at[slot]).wait()

    def mul_scatter(c, slot):
        def _mrow(r, _):
            for c16 in range(F // 32):
                gw = gates[slot][r, pl.ds(c16 * 16, 16)]
                ge = lax.bitcast_convert_type(gw << 16, jnp.float32)
                go = lax.bitcast_convert_type(gw & jnp.int32(-65536),
                                              jnp.float32)
                sa = pl.ds(c16 * 32, 16)
                sb = pl.ds(c16 * 32 + 16, 16)
                prods[slot][r, sa] = rows[slot][r, sa] * ge
                prods[slot][r, sb] = rows[slot][r, sb] * go
            return 0
        lax.fori_loop(0, CHUNK, _mrow, 0)
        pltpu.async_copy(prods[slot], msg_sh.at[idxi_v.at[slot]], ssem.at[slot],
                         add=True)

    def wait_scatter(c, slot):
        pltpu.make_async_copy(prods[slot], msg_sh.at[idxi_v.at[slot]],
                              ssem.at[slot]).wait()

    issue_idx(0, 0)
    issue_idx(1, 1)
    wait_idx(0, 0)
    issue_loads(0, 0)

    def _outer(i, _):
        for b in range(3):
            c = 3 * i + b
            if b == 0:
                @pl.when(c > 0)
                def _():
                    wait_scatter(c - 1, 2)
            else:
                wait_scatter(c - 1, (b + 2) % 3)

            @pl.when(c + 2 < N_CHUNKS)
            def _():
                issue_idx(c + 2, (b + 2) % 3)
            wait_idx(c + 1, (b + 1) % 3)
            issue_loads(c + 1, (b + 1) % 3)
            wait_loads(c, b)
            mul_scatter(c, b)
        return 0
    lax.fori_loop(0, (N_CHUNKS - 1) // 3, _outer, 0)

    tc = N_CHUNKS - 1
    wait_scatter(tc - 1, 2)
    wait_loads(tc, 0)
    mul_scatter(tc, 0)
    wait_scatter(tc, 0)

    plsc.subcore_barrier()
    pltpu.sync_copy(msg_sh.at[pl.ds(sid * ROWS_PER_S, ROWS_PER_S)],
                    out_hbm.at[cid, pl.ds(sid * ROWS_PER_S, ROWS_PER_S)])


_SC_CALL_CACHE = []


def _sc_msg_call(xj, gate, idx_j, idx_i):
    if not _SC_CALL_CACHE:
        _SC_CALL_CACHE.append(functools.partial(
            pl.kernel,
            out_type=jax.ShapeDtypeStruct((NC, N_PAD, F), jnp.float32),
            mesh=plsc.VectorSubcoreMesh(core_axis_name="c",
                                        subcore_axis_name="s",
                                        num_cores=NC, num_subcores=NS),
            scratch_types=[
                pltpu.VMEM((3, CHUNK), jnp.int32),
                pltpu.VMEM((3, CHUNK), jnp.int32),
                pltpu.VMEM((CHUNK, F), jnp.float32),
                pltpu.VMEM((CHUNK, F), jnp.float32),
                pltpu.VMEM((CHUNK, F), jnp.float32),
                pltpu.VMEM((CHUNK, F // 2), jnp.int32),
                pltpu.VMEM((CHUNK, F // 2), jnp.int32),
                pltpu.VMEM((CHUNK, F // 2), jnp.int32),
                pltpu.VMEM((CHUNK, F), jnp.float32),
                pltpu.VMEM((CHUNK, F), jnp.float32),
                pltpu.VMEM((CHUNK, F), jnp.float32),
                pltpu.VMEM((ZROWS, F), jnp.float32),
                pltpu.VMEM_SHARED((N_PAD, F), jnp.float32),
                pltpu.SemaphoreType.DMA((3,)),
                pltpu.SemaphoreType.DMA((3,)),
                pltpu.SemaphoreType.DMA((3,)),
                pltpu.SemaphoreType.DMA((3,)),
                pltpu.SemaphoreType.DMA((3,)),
            ],
        )(_sc_msg_body))
    return _SC_CALL_CACHE[0](xj, gate, idx_j, idx_i)


def kernel(atomic_numbers, Rij, idx_i, idx_j, emb, Wi, bi, Wj, bj, G, u,
           Wm, bm, ri_W1, ri_b1, ri_W2, ri_b2, ra_W1, ra_b1, ra_W2, ra_b2,
           ro_W1, ro_b1, ro_W2, ro_b2, Wo, bo):
    an = jnp.pad(atomic_numbers.astype(jnp.int32),
                 (0, N_PAD - N_ATOMS)).reshape(N_PAD, 1)
    emb_p = jnp.pad(emb.astype(jnp.float32), ((0, Z_PAD - emb.shape[0]), (0, 0)))
    idx_i = idx_i.astype(jnp.int32)
    idx_j = idx_j.astype(jnp.int32)
    rij = Rij.astype(jnp.float32)
    rx = rij[:, 0].reshape(1, N_EDGES)
    ry = rij[:, 1].reshape(1, N_EDGES)
    rz = rij[:, 2].reshape(1, N_EDGES)
    gt = _rbf_call(rx, ry, rz)

    g_lo = jnp.einsum("mij,jk->mik", G, jnp.asarray(_SL_NP))
    g_hi = jnp.einsum("mij,jk->mik", G, jnp.asarray(_SH_NP))
    x, xj = _embed_call(an, emb_p, Wj[0], bj[0].reshape(1, F))
    summ = jnp.zeros((N_PAD, F), jnp.float32)
    for m in range(N_MODULES):
        gate = _gate_call(gt, g_lo[m], g_hi[m])
        msgp = _sc_msg_call(xj, gate, idx_j, idx_i)
        args = (x, msgp, summ, Wi[m], bi[m].reshape(1, F),
                ri_W1[m], ri_b1[m], ri_W2[m], ri_b2[m],
                u[m].reshape(1, F), Wm[m], bm[m].reshape(1, F),
                ra_W1[m], ra_b1[m], ra_W2[m], ra_b2[m],
                ro_W1[m], ro_b1[m], ro_W2[m], ro_b2[m],
                Wo[m], bo[m].reshape(1, F))
        if m + 1 < N_MODULES:
            x, summ, xj = _nodeb_call(True, *args,
                                      wjn=Wj[m + 1], bjn=bj[m + 1].reshape(1, F))
        else:
            x, summ = _nodeb_call(False, *args)
    return summ[:N_ATOMS]
